# baseline (speedup 1.0000x reference)
import functools

import jax
import jax.numpy as jnp
from jax import lax
from jax.experimental import pallas as pl
from jax.experimental.pallas import tpu as pltpu
from jax.experimental.pallas import tpu_sc as plsc

_NC, _NS = 2, 16
_NW = _NC * _NS


def _make_gather(rows, n, h):
    b_per_w = rows // _NW
    mesh = plsc.VectorSubcoreMesh(core_axis_name="c", subcore_axis_name="s")

    @functools.partial(
        pl.kernel,
        mesh=mesh,
        out_type=jax.ShapeDtypeStruct((rows, h), jnp.float32),
        scratch_types=[
            pltpu.VMEM((b_per_w,), jnp.int32),
            pltpu.VMEM((b_per_w, h), jnp.float32),
            pltpu.SemaphoreType.DMA,
        ],
    )
    def gather_rows(table_hbm, idx_hbm, out_hbm, idx_v, rows_v, sem):
        wid = lax.axis_index("s") * _NC + lax.axis_index("c")
        base = wid * b_per_w
        pltpu.sync_copy(idx_hbm.at[pl.ds(base, b_per_w)], idx_v)
        flat = idx_v[...] + (base + lax.iota(jnp.int32, b_per_w)) * n
        idx_v[...] = flat
        pltpu.async_copy(table_hbm.at[idx_v], rows_v, sem).wait()
        pltpu.sync_copy(rows_v, out_hbm.at[pl.ds(base, b_per_w)])

    return gather_rows


def _mlp_body(pair_ref, w1_ref, b1_ref, w2_ref, b2_ref, out_ref, acc_ref, *,
              out_dim, n_steps):
    i = pl.program_id(0)

    @pl.when(i == 0)
    def _init():
        acc_ref[...] = jnp.zeros_like(acc_ref)

    h_blk = jnp.maximum(
        jnp.dot(pair_ref[...], w1_ref[...],
                preferred_element_type=jnp.float32) + b1_ref[...], 0.0)
    acc_ref[...] += jnp.dot(h_blk, w2_ref[...],
                            preferred_element_type=jnp.float32)

    @pl.when(i == n_steps - 1)
    def _finish():
        logits = acc_ref[...] + b2_ref[...]
        valid = lax.broadcasted_iota(jnp.int32, logits.shape, 1) < out_dim
        masked = jnp.where(valid, logits, -jnp.inf)
        m = jnp.max(masked, axis=1, keepdims=True)
        e = jnp.where(valid, jnp.exp(masked - m), 0.0)
        out_ref[...] = e / jnp.sum(e, axis=1, keepdims=True)


def kernel(directed_graph_encodings, previous_ids, W1, b1, W2, b2):
    b, n, _, h = directed_graph_encodings.shape
    out_dim = W2.shape[1]
    rows = b * n
    table = directed_graph_encodings.reshape(rows * n, h)
    idx = previous_ids.reshape(rows).astype(jnp.int32)
    pair = _make_gather(rows, n, h)(table, idx)
    w2p = jnp.pad(W2, ((0, 0), (0, 128 - out_dim)))
    b2p = jnp.pad(b2, (0, 128 - out_dim)).reshape(1, 128)
    n_steps = 4
    nb = h // n_steps
    probs = pl.pallas_call(
        functools.partial(_mlp_body, out_dim=out_dim, n_steps=n_steps),
        grid=(n_steps,),
        in_specs=[
            pl.BlockSpec((rows, h), lambda i: (0, 0)),
            pl.BlockSpec((h, nb), lambda i: (0, i)),
            pl.BlockSpec((1, nb), lambda i: (0, i)),
            pl.BlockSpec((nb, 128), lambda i: (i, 0)),
            pl.BlockSpec((1, 128), lambda i: (0, 0)),
        ],
        out_specs=pl.BlockSpec((rows, 128), lambda i: (0, 0)),
        scratch_shapes=[pltpu.VMEM((rows, 128), jnp.float32)],
        out_shape=jax.ShapeDtypeStruct((rows, 128), jnp.float32),
    )(pair, W1, b1.reshape(1, h), w2p, b2p)
    return probs.reshape(b, n, 128)[:, 1:, :out_dim]


# speedup vs baseline: 1.0424x; 1.0424x over previous
import functools

import jax
import jax.numpy as jnp
from jax import lax
from jax.experimental import pallas as pl
from jax.experimental.pallas import tpu as pltpu
from jax.experimental.pallas import tpu_sc as plsc

_NC, _NS = 2, 16
_NW = _NC * _NS


def _make_gather(rows, n, h):
    b_per_w = rows // _NW
    mesh = plsc.VectorSubcoreMesh(core_axis_name="c", subcore_axis_name="s")

    @functools.partial(
        pl.kernel,
        mesh=mesh,
        out_type=jax.ShapeDtypeStruct((rows, h), jnp.float32),
        scratch_types=[
            pltpu.VMEM((b_per_w,), jnp.int32),
            pltpu.VMEM((b_per_w, h), jnp.float32),
            pltpu.SemaphoreType.DMA,
        ],
    )
    def gather_rows(table_hbm, idx_hbm, out_hbm, idx_v, rows_v, sem):
        wid = lax.axis_index("s") * _NC + lax.axis_index("c")
        base = wid * b_per_w
        pltpu.sync_copy(idx_hbm.at[pl.ds(base, b_per_w)], idx_v)
        flat = idx_v[...] + (base + lax.iota(jnp.int32, b_per_w)) * n
        idx_v[...] = flat
        pltpu.async_copy(table_hbm.at[idx_v], rows_v, sem).wait()
        pltpu.sync_copy(rows_v, out_hbm.at[pl.ds(base, b_per_w)])

    return gather_rows


def _mlp_body(pair_ref, w1_ref, b1_ref, w2_ref, b2_ref, out_ref, *, out_dim):
    hidden = jnp.maximum(
        jnp.dot(pair_ref[...], w1_ref[...],
                preferred_element_type=jnp.float32) + b1_ref[...], 0.0)
    logits = jnp.dot(hidden, w2_ref[...],
                     preferred_element_type=jnp.float32) + b2_ref[...]
    valid = lax.broadcasted_iota(jnp.int32, logits.shape, 1) < out_dim
    masked = jnp.where(valid, logits, -jnp.inf)
    m = jnp.max(masked, axis=1, keepdims=True)
    e = jnp.where(valid, jnp.exp(masked - m), 0.0)
    probs = e / jnp.sum(e, axis=1, keepdims=True)
    b, n1, _ = out_ref.shape
    n = n1 + 1
    probs4 = probs.reshape(b, n, 128)
    out_ref[...] = probs4[:, 1:, :out_dim]


def kernel(directed_graph_encodings, previous_ids, W1, b1, W2, b2):
    b, n, _, h = directed_graph_encodings.shape
    out_dim = W2.shape[1]
    rows = b * n
    table = directed_graph_encodings.reshape(rows * n, h)
    idx = previous_ids.reshape(rows).astype(jnp.int32)
    pair = _make_gather(rows, n, h)(table, idx)
    w2p = jnp.pad(W2, ((0, 0), (0, 128 - out_dim)))
    b2p = jnp.pad(b2, (0, 128 - out_dim)).reshape(1, 128)
    return pl.pallas_call(
        functools.partial(_mlp_body, out_dim=out_dim),
        out_shape=jax.ShapeDtypeStruct((b, n - 1, out_dim), jnp.float32),
    )(pair, W1, b1.reshape(1, h), w2p, b2p)


# speedup vs baseline: 1.0503x; 1.0075x over previous
import functools

import jax
import jax.numpy as jnp
from jax import lax
from jax.experimental import pallas as pl
from jax.experimental.pallas import tpu as pltpu
from jax.experimental.pallas import tpu_sc as plsc

_NC, _NS = 2, 16
_NW = _NC * _NS


def _make_gather(rows, n, h):
    b_per_w = rows // _NS
    mesh = plsc.VectorSubcoreMesh(core_axis_name="c", subcore_axis_name="s",
                                  num_cores=1)

    @functools.partial(
        pl.kernel,
        mesh=mesh,
        out_type=jax.ShapeDtypeStruct((rows, h), jnp.float32),
        scratch_types=[
            pltpu.VMEM((b_per_w,), jnp.int32),
            pltpu.VMEM((b_per_w, h), jnp.float32),
            pltpu.SemaphoreType.DMA,
        ],
    )
    def gather_rows(table_hbm, idx_hbm, out_hbm, idx_v, rows_v, sem):
        wid = lax.axis_index("s")
        base = wid * b_per_w
        pltpu.sync_copy(idx_hbm.at[pl.ds(base, b_per_w)], idx_v)
        for c in range(b_per_w // 16):
            sl = pl.ds(c * 16, 16)
            idx_v[sl] = (idx_v[sl]
                         + (base + c * 16 + lax.iota(jnp.int32, 16)) * n)
        pltpu.async_copy(table_hbm.at[idx_v], rows_v, sem).wait()
        pltpu.sync_copy(rows_v, out_hbm.at[pl.ds(base, b_per_w)])

    return gather_rows


def _mlp_body(pair_ref, w1_ref, b1_ref, w2_ref, b2_ref, out_ref, *, out_dim):
    hidden = jnp.maximum(
        jnp.dot(pair_ref[...], w1_ref[...],
                preferred_element_type=jnp.float32) + b1_ref[...], 0.0)
    logits = jnp.dot(hidden, w2_ref[...],
                     preferred_element_type=jnp.float32) + b2_ref[...]
    valid = lax.broadcasted_iota(jnp.int32, logits.shape, 1) < out_dim
    masked = jnp.where(valid, logits, -jnp.inf)
    m = jnp.max(masked, axis=1, keepdims=True)
    e = jnp.where(valid, jnp.exp(masked - m), 0.0)
    probs = e / jnp.sum(e, axis=1, keepdims=True)
    b, n1, _ = out_ref.shape
    n = n1 + 1
    probs4 = probs.reshape(b, n, 128)
    out_ref[...] = probs4[:, 1:, :out_dim]


def kernel(directed_graph_encodings, previous_ids, W1, b1, W2, b2):
    b, n, _, h = directed_graph_encodings.shape
    out_dim = W2.shape[1]
    rows = b * n
    table = directed_graph_encodings.reshape(rows * n, h)
    idx = previous_ids.reshape(rows).astype(jnp.int32)
    pair = _make_gather(rows, n, h)(table, idx)
    w2p = jnp.pad(W2, ((0, 0), (0, 128 - out_dim)))
    b2p = jnp.pad(b2, (0, 128 - out_dim)).reshape(1, 128)
    return pl.pallas_call(
        functools.partial(_mlp_body, out_dim=out_dim),
        out_shape=jax.ShapeDtypeStruct((b, n - 1, out_dim), jnp.float32),
    )(pair, W1, b1.reshape(1, h), w2p, b2p)
